# baseline (speedup 1.0000x reference)
import functools

import jax
import jax.numpy as jnp
from jax import lax
from jax.experimental import pallas as pl
from jax.experimental.pallas import tpu as pltpu
from jax.experimental.pallas import tpu_sc as plsc

NEG_SLOPE = 0.2
EPS = 1e-16

NC = 2
NS = 16
BE = 80
RZB = 128
NP = 10240


def _logits_body(x_ref, af_ref, logits_ref, mx_ref):
    lg = jnp.dot(x_ref[...], af_ref[...], preferred_element_type=jnp.float32)
    logits_ref[...] = lg
    mx_ref[...] = jnp.max(lg, axis=0, keepdims=True)


def _logits(x, af):
    n = x.shape[0]
    k = af.shape[1]
    return pl.pallas_call(
        _logits_body,
        out_shape=(
            jax.ShapeDtypeStruct((n, k), jnp.float32),
            jax.ShapeDtypeStruct((1, k), jnp.float32),
        ),
    )(x, af)


def _proj1_body(xagg_ref, den_ref, w_ref, b_ref, out_ref):
    hh = xagg_ref.shape[0]
    c = xagg_ref.shape[2]
    den = den_ref[0] + den_ref[1]
    for h in range(hh):
        xn = xagg_ref[h] / (den[:, h:h + 1] + EPS)
        o = (jnp.dot(xn, w_ref[:, h * c:(h + 1) * c],
                     preferred_element_type=jnp.float32)
             + b_ref[:, h * c:(h + 1) * c])
        out_ref[:, h * c:(h + 1) * c] = jnp.where(
            o > 0, o, jnp.exp(jnp.minimum(o, 0.0)) - 1.0)


def _proj1(xagg, denp, w1, b1):
    hh, n, c = xagg.shape
    d = w1.shape[0]
    bn = 1024
    return pl.pallas_call(
        _proj1_body,
        grid=(n // bn,),
        in_specs=[
            pl.BlockSpec((hh, bn, c), lambda r: (0, r, 0)),
            pl.BlockSpec((NC, bn, 128), lambda r: (0, r, 0)),
            pl.BlockSpec((d, hh * c), lambda r: (0, 0)),
            pl.BlockSpec((1, hh * c), lambda r: (0, 0)),
        ],
        out_specs=pl.BlockSpec((bn, hh * c), lambda r: (r, 0)),
        out_shape=jax.ShapeDtypeStruct((n, hh * c), jnp.float32),
    )(xagg, denp, w1, b1.reshape(1, hh * c))


def _dense2_body(hmid_ref, w2_ref, af2_ref, h2_ref, lg2_ref, mx_ref):
    h2_ref[...] = jnp.dot(hmid_ref[...], w2_ref[...],
                          preferred_element_type=jnp.float32)
    lg = jnp.dot(hmid_ref[...], af2_ref[...],
                 preferred_element_type=jnp.float32)
    lg2_ref[...] = lg

    @pl.when(pl.program_id(0) == 0)
    def _():
        mx_ref[...] = jnp.full_like(mx_ref, -jnp.inf)

    mx_ref[...] = jnp.maximum(mx_ref[...], jnp.max(lg, axis=0, keepdims=True))


def _dense2(hmid, w2, af2):
    n, k = hmid.shape
    c = w2.shape[1]
    kk = af2.shape[1]
    bn = 2048
    return pl.pallas_call(
        _dense2_body,
        grid=(n // bn,),
        in_specs=[
            pl.BlockSpec((bn, k), lambda r: (r, 0)),
            pl.BlockSpec((k, c), lambda r: (0, 0)),
            pl.BlockSpec((k, kk), lambda r: (0, 0)),
        ],
        out_specs=(
            pl.BlockSpec((bn, c), lambda r: (r, 0)),
            pl.BlockSpec((bn, kk), lambda r: (r, 0)),
            pl.BlockSpec((1, kk), lambda r: (0, 0)),
        ),
        out_shape=(
            jax.ShapeDtypeStruct((n, c), jnp.float32),
            jax.ShapeDtypeStruct((n, kk), jnp.float32),
            jax.ShapeDtypeStruct((1, kk), jnp.float32),
        ),
    )(hmid, w2, af2)


def _final_body(agg_ref, den_ref, b_ref, out_ref):
    den = den_ref[0, :, 0:1] + den_ref[1, :, 0:1]
    out_ref[...] = (agg_ref[0] + agg_ref[1]) / (den + EPS) + b_ref[...]


def _final(aggp, denp, b2):
    _, n, c = aggp.shape
    return pl.pallas_call(
        _final_body,
        out_shape=jax.ShapeDtypeStruct((n, c), jnp.float32),
    )(aggp, denp, b2.reshape(1, c))


NBLK = 250


def _edge_logits_sc(tab, m16, s, d):
    e_total = s.shape[0]
    eb = e_total // (NC * NS)
    be = 40
    nblk = eb // be
    rpt = NP // NS
    mesh = plsc.VectorSubcoreMesh(core_axis_name="c", subcore_axis_name="s")

    buf_types = [
        pltpu.VMEM((be,), jnp.int32),
        pltpu.VMEM((be,), jnp.int32),
        pltpu.VMEM((be, 128), jnp.float32),
        pltpu.VMEM((be, 128), jnp.float32),
        pltpu.VMEM((be, 128), jnp.float32),
        pltpu.VMEM((be * 16,), jnp.float32),
        pltpu.SemaphoreType.DMA,
        pltpu.SemaphoreType.DMA,
    ]

    @functools.partial(
        pl.kernel, mesh=mesh,
        out_type=(
            jax.ShapeDtypeStruct((e_total * 16,), jnp.float32),
            jax.ShapeDtypeStruct((NC, NP, 128), jnp.float32),
        ),
        scratch_types=[
            pltpu.VMEM((16,), jnp.float32),
            pltpu.VMEM((RZB, 128), jnp.float32),
            pltpu.VMEM_SHARED((NP, 128), jnp.float32),
        ] + buf_types + buf_types,
    )
    def k(tab_h, m16_h, s_h, d_h, ee_h, den_h, m16_v, zb, den_acc, *bufs):
        c = lax.axis_index("c")
        sid = lax.axis_index("s")
        pltpu.sync_copy(m16_h, m16_v)
        k0, k1 = bufs[:8], bufs[8:]

        zv = jnp.zeros((16,), jnp.float32)

        def zrow(i, cc):
            for j in range(8):
                zb[i, pl.ds(j * 16, 16)] = zv
            return cc
        lax.fori_loop(0, RZB, zrow, 0)

        for K in (k0, k1):
            def zrow2(i, cc, eev2=K[4]):
                for j in range(8):
                    eev2[i, pl.ds(j * 16, 16)] = zv
                return cc
            lax.fori_loop(0, be, zrow2, 0)

        for kk in range(rpt // RZB):
            pltpu.sync_copy(zb, den_acc.at[pl.ds(sid * rpt + kk * RZB, RZB)])
        plsc.subcore_barrier()

        base = (c * NS + sid) * eb

        def do_idx(b, K):
            off = base + b * be
            pltpu.async_copy(s_h.at[pl.ds(off, be)], K[0], K[7])
            pltpu.async_copy(d_h.at[pl.ds(off, be)], K[1], K[7])

        def idx_wait(b, K):
            off = base + b * be
            pltpu.make_async_copy(s_h.at[pl.ds(off, be)], K[0], K[7]).wait()
            pltpu.make_async_copy(d_h.at[pl.ds(off, be)], K[1], K[7]).wait()

        def gather_start(b, K):
            pltpu.async_copy(tab_h.at[K[0]], K[2], K[6])
            pltpu.async_copy(tab_h.at[K[1]], K[3], K[6])

        def gather_wait(b, K):
            pltpu.make_async_copy(tab_h.at[K[0]], K[2], K[6]).wait()
            pltpu.make_async_copy(tab_h.at[K[1]], K[3], K[6]).wait()

        def compute(b, K):
            _, _, rs, rd, eev2, ee1, _, _ = K
            mv = m16_v[...]

            def edge(i, c2):
                z = rs[i, pl.ds(0, 16)] + rd[i, pl.ds(16, 16)]
                z = jnp.where(z >= 0.0, z, z * NEG_SLOPE)
                ee = jnp.exp(z - mv)
                eev2[i, pl.ds(0, 16)] = ee
                ee1[pl.ds(i * 16, 16)] = ee
                return c2
            lax.fori_loop(0, be, edge, 0, unroll=2)

        def scatter_sync(b, K):
            off = base + b * be
            pltpu.sync_copy(K[5], ee_h.at[pl.ds(off * 16, be * 16)])
            pltpu.sync_copy(K[4], den_acc.at[K[1]], add=True)

        _pipeline(nblk, k0, k1, do_idx, idx_wait, gather_start,
                  gather_wait, compute, scatter_sync)

        plsc.subcore_barrier()
        pltpu.sync_copy(den_acc.at[pl.ds(sid * rpt, rpt)],
                        den_h.at[c, pl.ds(sid * rpt, rpt)])

    return k(tab, m16, s, d)


def _pipeline(nblk, k0, k1, do_idx, idx_wait, gather_start, gather_wait,
              compute, scatter_sync):
    do_idx(0, k0)
    idx_wait(0, k0)
    gather_start(0, k0)
    do_idx(1, k1)
    gather_wait(0, k0)
    compute(0, k0)
    idx_wait(1, k1)
    gather_start(1, k1)
    scatter_sync(0, k0)

    def body(b, K, Kn):
        do_idx(b + 1, Kn)
        gather_wait(b, K)
        compute(b, K)
        idx_wait(b + 1, Kn)
        gather_start(b + 1, Kn)
        scatter_sync(b, K)

    def pair(b2, cc):
        b = 2 * b2 + 1
        body(b, k1, k0)
        body(b + 1, k0, k1)
        return cc
    lax.fori_loop(0, (nblk - 2) // 2, pair, 0)

    if nblk % 2 == 1:
        body(nblk - 2, k1, k0)
        klast = k0
    else:
        klast = k1
    gather_wait(nblk - 1, klast)
    compute(nblk - 1, klast)
    scatter_sync(nblk - 1, klast)


def _edge_agg_sc(ee1d, s, d, feat, heads):
    e_total = s.shape[0]
    n, cw = feat.shape
    n_pass = 4 if heads == 8 else 1
    edge_share = e_total // NS if heads == 8 else e_total // (NC * NS)
    be = 80 if heads == 8 else 40
    nblk = edge_share // be
    rpt = NP // NS
    mesh = plsc.VectorSubcoreMesh(core_axis_name="c", subcore_axis_name="s")
    out_major = heads if heads == 8 else NC

    buf_types = [
        pltpu.VMEM((be,), jnp.int32),
        pltpu.VMEM((be,), jnp.int32),
        pltpu.VMEM((be * 16,), jnp.float32),
        pltpu.VMEM((be, cw), jnp.float32),
        pltpu.SemaphoreType.DMA,
        pltpu.SemaphoreType.DMA,
    ]

    @functools.partial(
        pl.kernel, mesh=mesh,
        out_type=jax.ShapeDtypeStruct((out_major, NP, cw), jnp.float32),
        scratch_types=[
            pltpu.VMEM((RZB, cw), jnp.float32),
            pltpu.VMEM_SHARED((NP, cw), jnp.float32),
        ] + buf_types + buf_types,
    )
    def k(ee_h, s_h, d_h, feat_h, out_h, zb, acc, *bufs):
        c = lax.axis_index("c")
        sid = lax.axis_index("s")
        k0, k1 = bufs[:6], bufs[6:]

        def zrow(i, cc):
            for j in range(cw // 16):
                zb[i, pl.ds(j * 16, 16)] = jnp.zeros((16,), jnp.float32)
            return cc
        lax.fori_loop(0, RZB, zrow, 0)

        if heads == 8:
            base = sid * edge_share
        else:
            base = (c * NS + sid) * edge_share

        def do_idx(b, K):
            off = base + b * be
            pltpu.async_copy(s_h.at[pl.ds(off, be)], K[0], K[5])
            pltpu.async_copy(d_h.at[pl.ds(off, be)], K[1], K[5])
            pltpu.async_copy(ee_h.at[pl.ds(off * 16, be * 16)], K[2], K[5])

        def idx_wait(b, K):
            off = base + b * be
            pltpu.make_async_copy(s_h.at[pl.ds(off, be)], K[0], K[5]).wait()
            pltpu.make_async_copy(d_h.at[pl.ds(off, be)], K[1], K[5]).wait()
            pltpu.make_async_copy(
                ee_h.at[pl.ds(off * 16, be * 16)], K[2], K[5]).wait()

        def gather_start(b, K):
            pltpu.async_copy(feat_h.at[K[0]], K[3], K[4])

        def gather_wait(b, K):
            pltpu.make_async_copy(feat_h.at[K[0]], K[3], K[4]).wait()

        def scatter_sync(b, K):
            pltpu.sync_copy(K[3], acc.at[K[1]], add=True)

        for p in range(n_pass):
            lane = 4 * c + p if heads == 8 else 0
            lane_vec = jnp.full((16,), lane, jnp.int32)
            for kk in range(rpt // RZB):
                pltpu.sync_copy(zb, acc.at[pl.ds(sid * rpt + kk * RZB, RZB)])
            plsc.subcore_barrier()

            def compute(b, K, lane_vec=lane_vec):
                _, _, eev, rows, _, _ = K

                def edge(i, c2):
                    v = eev[pl.ds(i * 16, 16)]
                    wv = v.at[lane_vec].get(mode='promise_in_bounds')
                    for j in range(cw // 16):
                        rows[i, pl.ds(j * 16, 16)] = (
                            rows[i, pl.ds(j * 16, 16)] * wv)
                    return c2
                lax.fori_loop(0, be, edge, 0, unroll=4)

            _pipeline(nblk, k0, k1, do_idx, idx_wait, gather_start,
                      gather_wait, compute, scatter_sync)

            plsc.subcore_barrier()
            omaj = lane if heads == 8 else c
            pltpu.sync_copy(acc.at[pl.ds(sid * rpt, rpt)],
                            out_h.at[omaj, pl.ds(sid * rpt, rpt)])

    return k(ee1d, s, d, feat)


def kernel(x, edge_index, W1, a_src1, a_dst1, b1, W2, a_src2, a_dst2, b2):
    n, dd = x.shape
    hh = a_src1.shape[0]
    c = a_src1.shape[1]
    s = edge_index[0]
    d = edge_index[1]

    w1r = W1.reshape(dd, hh, c)
    afs1 = jnp.einsum('dhc,hc->dh', w1r, a_src1)
    afd1 = jnp.einsum('dhc,hc->dh', w1r, a_dst1)
    af1 = jnp.concatenate(
        [afs1, afs1, afd1, afd1, jnp.zeros((dd, 96), jnp.float32)], axis=1)
    afs2 = (W2 @ a_src2[0])[:, None]
    afd2 = (W2 @ a_dst2[0])[:, None]
    af2 = jnp.concatenate(
        [jnp.tile(afs2, (1, 16)), jnp.tile(afd2, (1, 16)),
         jnp.zeros((hh * c, 96), jnp.float32)], axis=1)

    lg1, mx1 = _logits(x, af1)
    m16_1 = mx1[0, :16] + mx1[0, 16:32]
    ee1, denp1 = _edge_logits_sc(lg1, m16_1, s, d)
    xagg = _edge_agg_sc(ee1, s, d, x, hh)
    hmid = _proj1(xagg, denp1, W1, b1)

    h2, lg2, mx2 = _dense2(hmid, W2, af2)
    m16_2 = mx2[0, :16] + mx2[0, 16:32]
    ee2, denp2 = _edge_logits_sc(lg2, m16_2, s, d)
    aggp2 = _edge_agg_sc(ee2, s, d, h2, 1)
    return _final(aggp2, denp2, b2)[:n]


# speedup vs baseline: 1.7575x; 1.7575x over previous
import functools

import jax
import jax.numpy as jnp
from jax import lax
from jax.experimental import pallas as pl
from jax.experimental.pallas import tpu as pltpu
from jax.experimental.pallas import tpu_sc as plsc

NEG_SLOPE = 0.2
EPS = 1e-16

NC = 2
NS = 16
BE = 80
RZB = 128
NP = 10240


def _logits_body(x_ref, af_ref, logits_ref, mx_ref):
    lg = jnp.dot(x_ref[...], af_ref[...], preferred_element_type=jnp.float32)
    logits_ref[...] = lg
    mx_ref[...] = jnp.max(lg, axis=0, keepdims=True)


def _logits(x, af):
    n = x.shape[0]
    k = af.shape[1]
    return pl.pallas_call(
        _logits_body,
        out_shape=(
            jax.ShapeDtypeStruct((n, k), jnp.float32),
            jax.ShapeDtypeStruct((1, k), jnp.float32),
        ),
    )(x, af)


def _proj1_body(xagg_ref, den_ref, w_ref, b_ref, out_ref):
    hh = xagg_ref.shape[0]
    c = xagg_ref.shape[2]
    den = den_ref[0] + den_ref[1]
    for h in range(hh):
        xn = xagg_ref[h] / (den[:, h:h + 1] + EPS)
        o = (jnp.dot(xn, w_ref[:, h * c:(h + 1) * c],
                     preferred_element_type=jnp.float32)
             + b_ref[:, h * c:(h + 1) * c])
        out_ref[:, h * c:(h + 1) * c] = jnp.where(
            o > 0, o, jnp.exp(jnp.minimum(o, 0.0)) - 1.0)


def _proj1(xagg, denp, w1, b1):
    hh, n, c = xagg.shape
    d = w1.shape[0]
    bn = 1024
    return pl.pallas_call(
        _proj1_body,
        grid=(n // bn,),
        in_specs=[
            pl.BlockSpec((hh, bn, c), lambda r: (0, r, 0)),
            pl.BlockSpec((NC, bn, 128), lambda r: (0, r, 0)),
            pl.BlockSpec((d, hh * c), lambda r: (0, 0)),
            pl.BlockSpec((1, hh * c), lambda r: (0, 0)),
        ],
        out_specs=pl.BlockSpec((bn, hh * c), lambda r: (r, 0)),
        out_shape=jax.ShapeDtypeStruct((n, hh * c), jnp.float32),
    )(xagg, denp, w1, b1.reshape(1, hh * c))


def _dense2_body(hmid_ref, w2_ref, af2_ref, h2_ref, lg2_ref, mx_ref):
    h2_ref[...] = jnp.dot(hmid_ref[...], w2_ref[...],
                          preferred_element_type=jnp.float32)
    lg = jnp.dot(hmid_ref[...], af2_ref[...],
                 preferred_element_type=jnp.float32)
    lg2_ref[...] = lg

    @pl.when(pl.program_id(0) == 0)
    def _():
        mx_ref[...] = jnp.full_like(mx_ref, -jnp.inf)

    mx_ref[...] = jnp.maximum(mx_ref[...], jnp.max(lg, axis=0, keepdims=True))


def _dense2(hmid, w2, af2):
    n, k = hmid.shape
    c = w2.shape[1]
    kk = af2.shape[1]
    bn = 2048
    return pl.pallas_call(
        _dense2_body,
        grid=(n // bn,),
        in_specs=[
            pl.BlockSpec((bn, k), lambda r: (r, 0)),
            pl.BlockSpec((k, c), lambda r: (0, 0)),
            pl.BlockSpec((k, kk), lambda r: (0, 0)),
        ],
        out_specs=(
            pl.BlockSpec((bn, c), lambda r: (r, 0)),
            pl.BlockSpec((bn, kk), lambda r: (r, 0)),
            pl.BlockSpec((1, kk), lambda r: (0, 0)),
        ),
        out_shape=(
            jax.ShapeDtypeStruct((n, c), jnp.float32),
            jax.ShapeDtypeStruct((n, kk), jnp.float32),
            jax.ShapeDtypeStruct((1, kk), jnp.float32),
        ),
    )(hmid, w2, af2)


def _final_body(agg_ref, den_ref, b_ref, out_ref):
    den = den_ref[0, :, 0:1] + den_ref[1, :, 0:1]
    out_ref[...] = (agg_ref[0] + agg_ref[1]) / (den + EPS) + b_ref[...]


def _final(aggp, denp, b2):
    _, n, c = aggp.shape
    return pl.pallas_call(
        _final_body,
        out_shape=jax.ShapeDtypeStruct((n, c), jnp.float32),
    )(aggp, denp, b2.reshape(1, c))


NBLK = 250


def _edge_logits_sc(tab, m16, s, d):
    e_total = s.shape[0]
    eb = e_total // (NC * NS)
    be = 40
    nblk = eb // be
    rpt = NP // NS
    mesh = plsc.VectorSubcoreMesh(core_axis_name="c", subcore_axis_name="s")

    buf_types = [
        pltpu.VMEM((be,), jnp.int32),
        pltpu.VMEM((be,), jnp.int32),
        pltpu.VMEM((be, 128), jnp.float32),
        pltpu.VMEM((be, 128), jnp.float32),
        pltpu.VMEM((be, 128), jnp.float32),
        pltpu.VMEM((be * 16,), jnp.float32),
        pltpu.SemaphoreType.DMA,
        pltpu.SemaphoreType.DMA,
    ]

    @functools.partial(
        pl.kernel, mesh=mesh,
        out_type=(
            jax.ShapeDtypeStruct((e_total * 16,), jnp.float32),
            jax.ShapeDtypeStruct((NC, NP, 128), jnp.float32),
        ),
        scratch_types=[
            pltpu.VMEM((16,), jnp.float32),
            pltpu.VMEM((RZB, 128), jnp.float32),
            pltpu.VMEM_SHARED((NP, 128), jnp.float32),
        ] + buf_types + buf_types,
    )
    def k(tab_h, m16_h, s_h, d_h, ee_h, den_h, m16_v, zb, den_acc, *bufs):
        c = lax.axis_index("c")
        sid = lax.axis_index("s")
        pltpu.sync_copy(m16_h, m16_v)
        k0, k1 = bufs[:8], bufs[8:]

        zv = jnp.zeros((16,), jnp.float32)

        def zrow(i, cc):
            for j in range(8):
                zb[i, pl.ds(j * 16, 16)] = zv
            return cc
        lax.fori_loop(0, RZB, zrow, 0)

        for K in (k0, k1):
            def zrow2(i, cc, eev2=K[4]):
                for j in range(8):
                    eev2[i, pl.ds(j * 16, 16)] = zv
                return cc
            lax.fori_loop(0, be, zrow2, 0)

        for kk in range(rpt // RZB):
            pltpu.sync_copy(zb, den_acc.at[pl.ds(sid * rpt + kk * RZB, RZB)])
        plsc.subcore_barrier()

        base = (c * NS + sid) * eb

        def do_idx(b, K):
            off = base + jnp.minimum(b, nblk - 1) * be
            pltpu.async_copy(s_h.at[pl.ds(off, be)], K[0], K[7])
            pltpu.async_copy(d_h.at[pl.ds(off, be)], K[1], K[7])

        def idx_wait(b, K):
            off = base + b * be
            pltpu.make_async_copy(s_h.at[pl.ds(off, be)], K[0], K[7]).wait()
            pltpu.make_async_copy(d_h.at[pl.ds(off, be)], K[1], K[7]).wait()

        def gather_start(b, K):
            pltpu.async_copy(tab_h.at[K[0]], K[2], K[6])
            pltpu.async_copy(tab_h.at[K[1]], K[3], K[6])

        def gather_wait(b, K):
            pltpu.make_async_copy(tab_h.at[K[0]], K[2], K[6]).wait()
            pltpu.make_async_copy(tab_h.at[K[1]], K[3], K[6]).wait()

        def compute(b, K):
            _, _, rs, rd, eev2, ee1, _, _ = K
            mv = m16_v[...]

            def edge(i, c2):
                z = rs[i, pl.ds(0, 16)] + rd[i, pl.ds(16, 16)]
                z = jnp.where(z >= 0.0, z, z * NEG_SLOPE)
                ee = jnp.exp(z - mv)
                eev2[i, pl.ds(0, 16)] = ee
                ee1[pl.ds(i * 16, 16)] = ee
                return c2
            lax.fori_loop(0, be, edge, 0, unroll=2)

        def scatter_sync(b, K):
            off = base + b * be
            pltpu.sync_copy(K[5], ee_h.at[pl.ds(off * 16, be * 16)])
            pltpu.sync_copy(K[4], den_acc.at[K[1]], add=True)

        _pipeline(nblk, k0, k1, do_idx, idx_wait, gather_start,
                  gather_wait, compute, scatter_sync)

        plsc.subcore_barrier()
        pltpu.sync_copy(den_acc.at[pl.ds(sid * rpt, rpt)],
                        den_h.at[c, pl.ds(sid * rpt, rpt)])

    return k(tab, m16, s, d)


def _pipeline(nblk, k0, k1, do_idx, idx_wait, gather_start, gather_wait,
              compute, scatter_sync):
    do_idx(0, k0)
    idx_wait(0, k0)
    gather_start(0, k0)
    do_idx(1, k1)
    idx_wait(1, k1)
    gather_start(1, k1)
    gather_wait(0, k0)
    compute(0, k0)
    scatter_sync(0, k0)
    do_idx(2, k0)

    def body(b, K, Kn):
        idx_wait(b + 1, Kn)
        gather_start(b + 1, Kn)
        gather_wait(b, K)
        compute(b, K)
        scatter_sync(b, K)
        do_idx(b + 2, K)

    def pair(b2, cc):
        b = 2 * b2 + 1
        body(b, k1, k0)
        body(b + 1, k0, k1)
        return cc
    lax.fori_loop(0, (nblk - 2) // 2, pair, 0)

    gather_wait(nblk - 1, k1)
    compute(nblk - 1, k1)
    scatter_sync(nblk - 1, k1)
    idx_wait(nblk - 1, k0)


def _edge_agg_sc(ee1d, s, d, feat, heads):
    e_total = s.shape[0]
    n, cw = feat.shape
    n_pass = 4 if heads == 8 else 1
    edge_share = e_total // NS if heads == 8 else e_total // (NC * NS)
    be = 80 if heads == 8 else 40
    nblk = edge_share // be
    rpt = NP // NS
    mesh = plsc.VectorSubcoreMesh(core_axis_name="c", subcore_axis_name="s")
    out_major = heads if heads == 8 else NC

    buf_types = [
        pltpu.VMEM((be,), jnp.int32),
        pltpu.VMEM((be,), jnp.int32),
        pltpu.VMEM((be * 16,), jnp.float32),
        pltpu.VMEM((be, cw), jnp.float32),
        pltpu.SemaphoreType.DMA,
        pltpu.SemaphoreType.DMA,
    ]

    @functools.partial(
        pl.kernel, mesh=mesh,
        out_type=jax.ShapeDtypeStruct((out_major, NP, cw), jnp.float32),
        scratch_types=[
            pltpu.VMEM((RZB, cw), jnp.float32),
            pltpu.VMEM_SHARED((NP, cw), jnp.float32),
        ] + buf_types + buf_types,
    )
    def k(ee_h, s_h, d_h, feat_h, out_h, zb, acc, *bufs):
        c = lax.axis_index("c")
        sid = lax.axis_index("s")
        k0, k1 = bufs[:6], bufs[6:]

        def zrow(i, cc):
            for j in range(cw // 16):
                zb[i, pl.ds(j * 16, 16)] = jnp.zeros((16,), jnp.float32)
            return cc
        lax.fori_loop(0, RZB, zrow, 0)

        if heads == 8:
            base = sid * edge_share
        else:
            base = (c * NS + sid) * edge_share

        def do_idx(b, K):
            off = base + jnp.minimum(b, nblk - 1) * be
            pltpu.async_copy(s_h.at[pl.ds(off, be)], K[0], K[5])
            pltpu.async_copy(d_h.at[pl.ds(off, be)], K[1], K[5])
            pltpu.async_copy(ee_h.at[pl.ds(off * 16, be * 16)], K[2], K[5])

        def idx_wait(b, K):
            off = base + b * be
            pltpu.make_async_copy(s_h.at[pl.ds(off, be)], K[0], K[5]).wait()
            pltpu.make_async_copy(d_h.at[pl.ds(off, be)], K[1], K[5]).wait()
            pltpu.make_async_copy(
                ee_h.at[pl.ds(off * 16, be * 16)], K[2], K[5]).wait()

        def gather_start(b, K):
            pltpu.async_copy(feat_h.at[K[0]], K[3], K[4])

        def gather_wait(b, K):
            pltpu.make_async_copy(feat_h.at[K[0]], K[3], K[4]).wait()

        def scatter_sync(b, K):
            pltpu.sync_copy(K[3], acc.at[K[1]], add=True)

        for p in range(n_pass):
            lane = 4 * c + p if heads == 8 else 0
            lane_vec = jnp.full((16,), lane, jnp.int32)
            for kk in range(rpt // RZB):
                pltpu.sync_copy(zb, acc.at[pl.ds(sid * rpt + kk * RZB, RZB)])
            plsc.subcore_barrier()

            def compute(b, K, lane_vec=lane_vec):
                _, _, eev, rows, _, _ = K

                def edge(i, c2):
                    v = eev[pl.ds(i * 16, 16)]
                    wv = v.at[lane_vec].get(mode='promise_in_bounds')
                    for j in range(cw // 16):
                        rows[i, pl.ds(j * 16, 16)] = (
                            rows[i, pl.ds(j * 16, 16)] * wv)
                    return c2
                lax.fori_loop(0, be, edge, 0, unroll=2)

            _pipeline(nblk, k0, k1, do_idx, idx_wait, gather_start,
                      gather_wait, compute, scatter_sync)

            plsc.subcore_barrier()
            omaj = lane if heads == 8 else c
            pltpu.sync_copy(acc.at[pl.ds(sid * rpt, rpt)],
                            out_h.at[omaj, pl.ds(sid * rpt, rpt)])

    return k(ee1d, s, d, feat)


def kernel(x, edge_index, W1, a_src1, a_dst1, b1, W2, a_src2, a_dst2, b2):
    n, dd = x.shape
    hh = a_src1.shape[0]
    c = a_src1.shape[1]
    s = edge_index[0]
    d = edge_index[1]

    w1r = W1.reshape(dd, hh, c)
    afs1 = jnp.einsum('dhc,hc->dh', w1r, a_src1)
    afd1 = jnp.einsum('dhc,hc->dh', w1r, a_dst1)
    af1 = jnp.concatenate(
        [afs1, afs1, afd1, afd1, jnp.zeros((dd, 96), jnp.float32)], axis=1)
    afs2 = (W2 @ a_src2[0])[:, None]
    afd2 = (W2 @ a_dst2[0])[:, None]
    af2 = jnp.concatenate(
        [jnp.tile(afs2, (1, 16)), jnp.tile(afd2, (1, 16)),
         jnp.zeros((hh * c, 96), jnp.float32)], axis=1)

    lg1, mx1 = _logits(x, af1)
    m16_1 = mx1[0, :16] + mx1[0, 16:32]
    ee1, denp1 = _edge_logits_sc(lg1, m16_1, s, d)
    xagg = _edge_agg_sc(ee1, s, d, x, hh)
    hmid = _proj1(xagg, denp1, W1, b1)

    h2, lg2, mx2 = _dense2(hmid, W2, af2)
    m16_2 = mx2[0, :16] + mx2[0, 16:32]
    ee2, denp2 = _edge_logits_sc(lg2, m16_2, s, d)
    aggp2 = _edge_agg_sc(ee2, s, d, h2, 1)
    return _final(aggp2, denp2, b2)[:n]


# speedup vs baseline: 1.7987x; 1.0235x over previous
import functools

import jax
import jax.numpy as jnp
from jax import lax
from jax.experimental import pallas as pl
from jax.experimental.pallas import tpu as pltpu
from jax.experimental.pallas import tpu_sc as plsc

NEG_SLOPE = 0.2
EPS = 1e-16

NC = 2
NS = 16
BE = 80
RZB = 128
NP = 10240


def _logits_body(x_ref, af_ref, logits_ref, mx_ref):
    lg = jnp.dot(x_ref[...], af_ref[...], preferred_element_type=jnp.float32)
    logits_ref[...] = lg
    mx_ref[...] = jnp.max(lg, axis=0, keepdims=True)


def _logits(x, af):
    n = x.shape[0]
    k = af.shape[1]
    return pl.pallas_call(
        _logits_body,
        out_shape=(
            jax.ShapeDtypeStruct((n, k), jnp.float32),
            jax.ShapeDtypeStruct((1, k), jnp.float32),
        ),
    )(x, af)


def _proj1_body(xagg_ref, den_ref, w_ref, b_ref, out_ref):
    hh = xagg_ref.shape[0]
    c = xagg_ref.shape[2]
    den = den_ref[0] + den_ref[1]
    for h in range(hh):
        xn = xagg_ref[h] / (den[:, h:h + 1] + EPS)
        o = (jnp.dot(xn, w_ref[:, h * c:(h + 1) * c],
                     preferred_element_type=jnp.float32)
             + b_ref[:, h * c:(h + 1) * c])
        out_ref[:, h * c:(h + 1) * c] = jnp.where(
            o > 0, o, jnp.exp(jnp.minimum(o, 0.0)) - 1.0)


def _proj1(xagg, denp, w1, b1):
    hh, n, c = xagg.shape
    d = w1.shape[0]
    bn = 1024
    return pl.pallas_call(
        _proj1_body,
        grid=(n // bn,),
        in_specs=[
            pl.BlockSpec((hh, bn, c), lambda r: (0, r, 0)),
            pl.BlockSpec((NC, bn, 128), lambda r: (0, r, 0)),
            pl.BlockSpec((d, hh * c), lambda r: (0, 0)),
            pl.BlockSpec((1, hh * c), lambda r: (0, 0)),
        ],
        out_specs=pl.BlockSpec((bn, hh * c), lambda r: (r, 0)),
        out_shape=jax.ShapeDtypeStruct((n, hh * c), jnp.float32),
    )(xagg, denp, w1, b1.reshape(1, hh * c))


def _dense2_body(hmid_ref, w2_ref, af2_ref, h2_ref, lg2_ref, mx_ref):
    h2_ref[...] = jnp.dot(hmid_ref[...], w2_ref[...],
                          preferred_element_type=jnp.float32)
    lg = jnp.dot(hmid_ref[...], af2_ref[...],
                 preferred_element_type=jnp.float32)
    lg2_ref[...] = lg

    @pl.when(pl.program_id(0) == 0)
    def _():
        mx_ref[...] = jnp.full_like(mx_ref, -jnp.inf)

    mx_ref[...] = jnp.maximum(mx_ref[...], jnp.max(lg, axis=0, keepdims=True))


def _dense2(hmid, w2, af2):
    n, k = hmid.shape
    c = w2.shape[1]
    kk = af2.shape[1]
    bn = 2048
    return pl.pallas_call(
        _dense2_body,
        grid=(n // bn,),
        in_specs=[
            pl.BlockSpec((bn, k), lambda r: (r, 0)),
            pl.BlockSpec((k, c), lambda r: (0, 0)),
            pl.BlockSpec((k, kk), lambda r: (0, 0)),
        ],
        out_specs=(
            pl.BlockSpec((bn, c), lambda r: (r, 0)),
            pl.BlockSpec((bn, kk), lambda r: (r, 0)),
            pl.BlockSpec((1, kk), lambda r: (0, 0)),
        ),
        out_shape=(
            jax.ShapeDtypeStruct((n, c), jnp.float32),
            jax.ShapeDtypeStruct((n, kk), jnp.float32),
            jax.ShapeDtypeStruct((1, kk), jnp.float32),
        ),
    )(hmid, w2, af2)


def _final_body(agg_ref, den_ref, b_ref, out_ref):
    den = den_ref[0, :, 0:1] + den_ref[1, :, 0:1]
    out_ref[...] = (agg_ref[0] + agg_ref[1]) / (den + EPS) + b_ref[...]


def _final(aggp, denp, b2):
    _, n, c = aggp.shape
    return pl.pallas_call(
        _final_body,
        out_shape=jax.ShapeDtypeStruct((n, c), jnp.float32),
    )(aggp, denp, b2.reshape(1, c))


NBLK = 250


def _edge_logits_sc(tab, m16, s, d):
    e_total = s.shape[0]
    eb = e_total // (NC * NS)
    be = 40
    nblk = eb // be
    rpt = NP // NS
    mesh = plsc.VectorSubcoreMesh(core_axis_name="c", subcore_axis_name="s")

    buf_types = [
        pltpu.VMEM((be,), jnp.int32),
        pltpu.VMEM((be,), jnp.int32),
        pltpu.VMEM((be, 128), jnp.float32),
        pltpu.VMEM((be, 128), jnp.float32),
        pltpu.VMEM((be * 16,), jnp.float32),
        pltpu.SemaphoreType.DMA,
        pltpu.SemaphoreType.DMA,
    ]

    @functools.partial(
        pl.kernel, mesh=mesh,
        out_type=(
            jax.ShapeDtypeStruct((e_total * 16,), jnp.float32),
            jax.ShapeDtypeStruct((NC, NP, 128), jnp.float32),
        ),
        scratch_types=[
            pltpu.VMEM((16,), jnp.float32),
            pltpu.VMEM((RZB, 128), jnp.float32),
            pltpu.VMEM_SHARED((NP, 128), jnp.float32),
        ] + buf_types + buf_types,
    )
    def k(tab_h, m16_h, s_h, d_h, ee_h, den_h, m16_v, zb, den_acc, *bufs):
        c = lax.axis_index("c")
        sid = lax.axis_index("s")
        pltpu.sync_copy(m16_h, m16_v)
        k0, k1 = bufs[:7], bufs[7:]

        zv = jnp.zeros((16,), jnp.float32)

        def zrow(i, cc):
            for j in range(8):
                zb[i, pl.ds(j * 16, 16)] = zv
            return cc
        lax.fori_loop(0, RZB, zrow, 0)

        for kk in range(rpt // RZB):
            pltpu.sync_copy(zb, den_acc.at[pl.ds(sid * rpt + kk * RZB, RZB)])
        plsc.subcore_barrier()

        base = (c * NS + sid) * eb

        def do_idx(b, K):
            off = base + jnp.minimum(b, nblk - 1) * be
            pltpu.async_copy(s_h.at[pl.ds(off, be)], K[0], K[6])
            pltpu.async_copy(d_h.at[pl.ds(off, be)], K[1], K[6])

        def idx_wait(b, K):
            off = base + jnp.minimum(b, nblk - 1) * be
            pltpu.make_async_copy(s_h.at[pl.ds(off, be)], K[0], K[6]).wait()
            pltpu.make_async_copy(d_h.at[pl.ds(off, be)], K[1], K[6]).wait()

        def gather_start(b, K):
            pltpu.async_copy(tab_h.at[K[0]], K[2], K[5])
            pltpu.async_copy(tab_h.at[K[1]], K[3], K[5])

        def gather_wait(b, K):
            pltpu.make_async_copy(tab_h.at[K[0]], K[2], K[5]).wait()
            pltpu.make_async_copy(tab_h.at[K[1]], K[3], K[5]).wait()

        def compute(b, K):
            _, _, rs, rd, ee1, _, _ = K
            mv = m16_v[...]

            def edge(i, c2):
                z = rs[i, pl.ds(0, 16)] + rd[i, pl.ds(16, 16)]
                z = jnp.where(z >= 0.0, z, z * NEG_SLOPE)
                ee = jnp.exp(z - mv)
                rs[i, pl.ds(0, 16)] = ee
                rs[i, pl.ds(16, 16)] = zv
                ee1[pl.ds(i * 16, 16)] = ee
                return c2
            lax.fori_loop(0, be, edge, 0, unroll=2)

        def scatter_sync(b, K):
            off = base + b * be
            pltpu.sync_copy(K[4], ee_h.at[pl.ds(off * 16, be * 16)])
            pltpu.sync_copy(K[2], den_acc.at[K[1]], add=True)

        _pipeline(nblk, k0, k1, do_idx, idx_wait, gather_start,
                  gather_wait, compute, scatter_sync)

        plsc.subcore_barrier()
        pltpu.sync_copy(den_acc.at[pl.ds(sid * rpt, rpt)],
                        den_h.at[c, pl.ds(sid * rpt, rpt)])

    return k(tab, m16, s, d)


def _pipeline(nblk, k0, k1, do_idx, idx_wait, gather_start, gather_wait,
              compute, scatter_sync):
    do_idx(0, k0)
    idx_wait(0, k0)
    gather_start(0, k0)
    do_idx(1, k1)
    idx_wait(1, k1)
    gather_start(1, k1)
    gather_wait(0, k0)
    compute(0, k0)
    scatter_sync(0, k0)
    do_idx(2, k0)

    def body(b, K, Kn):
        idx_wait(b + 1, Kn)
        gather_start(b + 1, Kn)
        gather_wait(b, K)
        compute(b, K)
        scatter_sync(b, K)
        do_idx(b + 2, K)

    def pair(b2, cc):
        b = 2 * b2 + 1
        body(b, k1, k0)
        body(b + 1, k0, k1)
        return cc
    lax.fori_loop(0, (nblk - 2) // 2, pair, 0)

    if nblk % 2 == 1:
        body(nblk - 2, k1, k0)
        klast, kdrain = k0, k1
    else:
        klast, kdrain = k1, k0
    gather_wait(nblk - 1, klast)
    compute(nblk - 1, klast)
    scatter_sync(nblk - 1, klast)
    idx_wait(nblk - 1, kdrain)


def _edge_agg_sc(ee1d, s, d, feat, heads):
    e_total = s.shape[0]
    n, cw = feat.shape
    n_pass = 4 if heads == 8 else 1
    edge_share = e_total // NS if heads == 8 else e_total // (NC * NS)
    be = 80
    nblk = edge_share // be
    rpt = NP // NS
    mesh = plsc.VectorSubcoreMesh(core_axis_name="c", subcore_axis_name="s")
    out_major = heads if heads == 8 else NC

    buf_types = [
        pltpu.VMEM((be,), jnp.int32),
        pltpu.VMEM((be,), jnp.int32),
        pltpu.VMEM((be * 16,), jnp.float32),
        pltpu.VMEM((be, cw), jnp.float32),
        pltpu.SemaphoreType.DMA,
        pltpu.SemaphoreType.DMA,
    ]

    @functools.partial(
        pl.kernel, mesh=mesh,
        out_type=jax.ShapeDtypeStruct((out_major, NP, cw), jnp.float32),
        scratch_types=[
            pltpu.VMEM((RZB, cw), jnp.float32),
            pltpu.VMEM_SHARED((NP, cw), jnp.float32),
        ] + buf_types + buf_types,
    )
    def k(ee_h, s_h, d_h, feat_h, out_h, zb, acc, *bufs):
        c = lax.axis_index("c")
        sid = lax.axis_index("s")
        k0, k1 = bufs[:6], bufs[6:]

        def zrow(i, cc):
            for j in range(cw // 16):
                zb[i, pl.ds(j * 16, 16)] = jnp.zeros((16,), jnp.float32)
            return cc
        lax.fori_loop(0, RZB, zrow, 0)

        if heads == 8:
            base = sid * edge_share
        else:
            base = (c * NS + sid) * edge_share

        def do_idx(b, K):
            off = base + jnp.minimum(b, nblk - 1) * be
            pltpu.async_copy(s_h.at[pl.ds(off, be)], K[0], K[5])
            pltpu.async_copy(d_h.at[pl.ds(off, be)], K[1], K[5])
            pltpu.async_copy(ee_h.at[pl.ds(off * 16, be * 16)], K[2], K[5])

        def idx_wait(b, K):
            off = base + b * be
            pltpu.make_async_copy(s_h.at[pl.ds(off, be)], K[0], K[5]).wait()
            pltpu.make_async_copy(d_h.at[pl.ds(off, be)], K[1], K[5]).wait()
            pltpu.make_async_copy(
                ee_h.at[pl.ds(off * 16, be * 16)], K[2], K[5]).wait()

        def gather_start(b, K):
            pltpu.async_copy(feat_h.at[K[0]], K[3], K[4])

        def gather_wait(b, K):
            pltpu.make_async_copy(feat_h.at[K[0]], K[3], K[4]).wait()

        def scatter_sync(b, K):
            pltpu.sync_copy(K[3], acc.at[K[1]], add=True)

        for p in range(n_pass):
            lane = 4 * c + p if heads == 8 else 0
            lane_vec = jnp.full((16,), lane, jnp.int32)
            for kk in range(rpt // RZB):
                pltpu.sync_copy(zb, acc.at[pl.ds(sid * rpt + kk * RZB, RZB)])
            plsc.subcore_barrier()

            def compute(b, K, lane_vec=lane_vec):
                _, _, eev, rows, _, _ = K

                def edge(i, c2):
                    v = eev[pl.ds(i * 16, 16)]
                    wv = v.at[lane_vec].get(mode='promise_in_bounds')
                    for j in range(cw // 16):
                        rows[i, pl.ds(j * 16, 16)] = (
                            rows[i, pl.ds(j * 16, 16)] * wv)
                    return c2
                lax.fori_loop(0, be, edge, 0, unroll=2)

            _pipeline(nblk, k0, k1, do_idx, idx_wait, gather_start,
                      gather_wait, compute, scatter_sync)

            plsc.subcore_barrier()
            omaj = lane if heads == 8 else c
            pltpu.sync_copy(acc.at[pl.ds(sid * rpt, rpt)],
                            out_h.at[omaj, pl.ds(sid * rpt, rpt)])

    return k(ee1d, s, d, feat)


def kernel(x, edge_index, W1, a_src1, a_dst1, b1, W2, a_src2, a_dst2, b2):
    n, dd = x.shape
    hh = a_src1.shape[0]
    c = a_src1.shape[1]
    s = edge_index[0]
    d = edge_index[1]

    w1r = W1.reshape(dd, hh, c)
    afs1 = jnp.einsum('dhc,hc->dh', w1r, a_src1)
    afd1 = jnp.einsum('dhc,hc->dh', w1r, a_dst1)
    af1 = jnp.concatenate(
        [afs1, afs1, afd1, afd1, jnp.zeros((dd, 96), jnp.float32)], axis=1)
    afs2 = (W2 @ a_src2[0])[:, None]
    afd2 = (W2 @ a_dst2[0])[:, None]
    af2 = jnp.concatenate(
        [jnp.tile(afs2, (1, 16)), jnp.tile(afd2, (1, 16)),
         jnp.zeros((hh * c, 96), jnp.float32)], axis=1)

    lg1, mx1 = _logits(x, af1)
    m16_1 = mx1[0, :16] + mx1[0, 16:32]
    ee1, denp1 = _edge_logits_sc(lg1, m16_1, s, d)
    xagg = _edge_agg_sc(ee1, s, d, x, hh)
    hmid = _proj1(xagg, denp1, W1, b1)

    h2, lg2, mx2 = _dense2(hmid, W2, af2)
    m16_2 = mx2[0, :16] + mx2[0, 16:32]
    ee2, denp2 = _edge_logits_sc(lg2, m16_2, s, d)
    aggp2 = _edge_agg_sc(ee2, s, d, h2, 1)
    return _final(aggp2, denp2, b2)[:n]
